# baseline (speedup 1.0000x reference)
import functools

import jax
import jax.numpy as jnp
from jax.experimental import pallas as pl

N = 2048
NFEAT = 256
NHID = 128
NCLASS = 16
RT = 256
NRT = N // RT


def _mlp_prep_kernel(x_ref, cW1_ref, cb1_ref, cW2r_ref, cb2_ref,
                     fW1_ref, fb1_ref, fW2r_ref, fb2_ref, gW0_ref,
                     kappa_ref, f_ref, h0_ref):
    x = x_ref[...]
    hc = jax.nn.relu(jnp.dot(x, cW1_ref[...],
                             preferred_element_type=jnp.float32) + cb1_ref[...])
    kappa = jax.nn.sigmoid(
        jnp.sum(hc * cW2r_ref[...], axis=1, keepdims=True) + cb2_ref[...])
    hf = jax.nn.relu(jnp.dot(x, fW1_ref[...],
                             preferred_element_type=jnp.float32) + fb1_ref[...])
    f = jnp.sum(hf * fW2r_ref[...], axis=1, keepdims=True) + fb2_ref[...]
    kappa_ref[...] = kappa
    f_ref[...] = f
    h0_ref[...] = jnp.dot(x, gW0_ref[...], preferred_element_type=jnp.float32)


def _rank_mask_kernel(kc_ref, kr8_ref, krm_ref, mask_ref):
    kc = kc_ref[...]
    ii = jax.lax.broadcasted_iota(jnp.int32, (N, RT), 0)

    def body(c, acc):
        krc = kr8_ref[pl.ds(c, 1), :]
        jj = jax.lax.broadcasted_iota(jnp.int32, (N, RT), 1) + c * RT
        gt = (krc > kc).astype(jnp.float32)
        tie = jnp.logical_and(krc == kc, jj < ii).astype(jnp.float32)
        return acc + jnp.sum(gt + tie, axis=1, keepdims=True)

    rank = jax.lax.fori_loop(0, NRT, body, jnp.zeros((N, 1), jnp.float32))
    mask_ref[...] = (rank >= krm_ref[...]).astype(jnp.float32)


def _pass_b_kernel(adj_ref, wts_ref, fr_ref, mr_ref, fc_ref, mc_ref, h0_ref,
                   s0_ref, s1_ref, g_ref, df_ref, d1_ref, d2_ref, g0_ref):
    adj = adj_ref[...]
    wa = adj * jax.nn.sigmoid(wts_ref[...])
    fr = fr_ref[...]
    fc = fc_ref[...]
    s0 = jnp.sum(wa, axis=1, keepdims=True)
    s1 = jnp.sum(wa * fr, axis=1, keepdims=True)
    s2 = jnp.sum(wa * (fr * fr), axis=1, keepdims=True)
    sa = jnp.sum(adj, axis=1, keepdims=True)
    sm = jnp.sum(adj * mr_ref[...], axis=1, keepdims=True)
    s0_ref[...] = s0
    s1_ref[...] = s1
    g_ref[...] = 0.5 * (s2 - 2.0 * fc * s1 + fc * fc * s0)
    df_ref[...] = s1 - fc * s0
    d1 = jax.lax.rsqrt(sa + 1.0)
    d1_ref[...] = d1
    d2_ref[...] = jax.lax.rsqrt(mc_ref[...] * sm + 1.0)
    g0_ref[...] = d1 * h0_ref[...]


def _pass_c_kernel(adj_ref, wts_ref, gr_ref, dfr_ref, fr_ref,
                   s0_ref, s1_ref, fc_ref, dfc_ref, gc_ref,
                   g0full_ref, d1_ref, gb0_ref, mc_ref, d2_ref, gW1_ref,
                   g2_ref, z1_ref, zm_ref):
    i = pl.program_id(0)
    adj = adj_ref[...]
    wa = adj * jax.nn.sigmoid(wts_ref[...])
    gr = gr_ref[...]
    dfr = dfr_ref[...]
    fr = fr_ref[...]
    s0 = s0_ref[...]
    s1 = s1_ref[...]
    fc = fc_ref[...]
    dfc = dfc_ref[...]
    t_g = jnp.sum(wa * gr, axis=1, keepdims=True)
    t_d = jnp.sum(wa * dfr, axis=1, keepdims=True)
    t_fd = jnp.sum(wa * (fr * dfr), axis=1, keepdims=True)
    delta_gamma = t_g - gc_ref[...] * s0
    gfd = 0.5 * (t_fd - fc * t_d - dfc * s1 + fc * dfc * s0)
    g2_ref[...] = 0.5 * delta_gamma - gfd
    y0 = jnp.dot(adj, g0full_ref[...], preferred_element_type=jnp.float32)
    g0_rows = g0full_ref[pl.ds(i * RT, RT), :]
    h1 = jax.nn.relu(d1_ref[...] * (y0 + g0_rows) + gb0_ref[...])
    z1 = d2_ref[...] * jnp.dot(h1, gW1_ref[...],
                               preferred_element_type=jnp.float32)
    z1_ref[...] = z1
    zm_ref[...] = mc_ref[...] * z1


def _pass_d_kernel(adj_ref, zmfull_ref, z1_ref, mc_ref, d2_ref, gb1_ref,
                   kc_ref, gr_ref, g2r_ref, out_ref, curv_ref):
    i = pl.program_id(0)
    adj = adj_ref[...]
    y = jnp.dot(adj, zmfull_ref[...], preferred_element_type=jnp.float32)
    agg = mc_ref[...] * y + z1_ref[...]
    h2 = jax.nn.relu(d2_ref[...] * agg + gb1_ref[...])
    m = jnp.max(h2, axis=1, keepdims=True)
    e = h2 - m
    out_ref[...] = e - jnp.log(jnp.sum(jnp.exp(e), axis=1, keepdims=True))
    part = jnp.reshape(
        jnp.sum(jax.nn.relu(kc_ref[...] * gr_ref[...] - g2r_ref[...])), (1, 1))

    @pl.when(i == 0)
    def _():
        curv_ref[...] = part

    @pl.when(i > 0)
    def _():
        curv_ref[...] += part


def _row(v):
    return jnp.reshape(v, (1, N))


@functools.partial(jax.jit, static_argnames=())
def kernel(x, adj, weights, cW1, cb1, cW2, cb2, fW1, fb1, fW2, fb2,
           gW0, gb0, gW1, gb1, p):
    f32 = jnp.float32
    col = lambda n: jax.ShapeDtypeStruct((n, 1), f32)
    rowblk = pl.BlockSpec((RT, N), lambda i: (i, 0))
    colblk = pl.BlockSpec((RT, 1), lambda i: (i, 0))
    full = lambda shape: pl.BlockSpec(shape, lambda i: (0, 0))

    kappa, f, h0 = pl.pallas_call(
        _mlp_prep_kernel,
        out_shape=(col(N), col(N), jax.ShapeDtypeStruct((N, NHID), f32)),
    )(x, cW1, jnp.reshape(cb1, (1, NHID)), jnp.reshape(cW2, (1, NHID)),
      jnp.reshape(cb2, (1, 1)), fW1, jnp.reshape(fb1, (1, NHID)),
      jnp.reshape(fW2, (1, NHID)), jnp.reshape(fb2, (1, 1)), gW0)

    k_rm = jnp.floor_divide(N * p, 100).astype(f32).reshape(1, 1)
    mask = pl.pallas_call(
        _rank_mask_kernel,
        out_shape=col(N),
    )(kappa, jnp.reshape(kappa, (NRT, RT)), k_rm)

    f_r, mask_r = _row(f), _row(mask)

    s0, s1, gamma_f, delta_f, dinv1, dinv2, g0 = pl.pallas_call(
        _pass_b_kernel,
        grid=(NRT,),
        in_specs=[rowblk, rowblk, full((1, N)), full((1, N)), colblk, colblk,
                  pl.BlockSpec((RT, NHID), lambda i: (i, 0))],
        out_specs=(colblk, colblk, colblk, colblk, colblk, colblk,
                   pl.BlockSpec((RT, NHID), lambda i: (i, 0))),
        out_shape=(col(N),) * 6 + (jax.ShapeDtypeStruct((N, NHID), f32),),
    )(adj, weights, f_r, mask_r, f, mask, h0)

    g_r, df_r = _row(gamma_f), _row(delta_f)

    gamma2, z1, zm = pl.pallas_call(
        _pass_c_kernel,
        grid=(NRT,),
        in_specs=[rowblk, rowblk, full((1, N)), full((1, N)), full((1, N)),
                  colblk, colblk, colblk, colblk, colblk,
                  full((N, NHID)), colblk, full((1, NHID)), colblk, colblk,
                  full((NHID, NCLASS))],
        out_specs=(colblk, pl.BlockSpec((RT, NCLASS), lambda i: (i, 0)),
                   pl.BlockSpec((RT, NCLASS), lambda i: (i, 0))),
        out_shape=(col(N), jax.ShapeDtypeStruct((N, NCLASS), f32),
                   jax.ShapeDtypeStruct((N, NCLASS), f32)),
    )(adj, weights, g_r, df_r, f_r, s0, s1, f, delta_f, gamma_f,
      g0, dinv1, jnp.reshape(gb0, (1, NHID)), mask, dinv2, gW1)

    out, curv = pl.pallas_call(
        _pass_d_kernel,
        grid=(NRT,),
        in_specs=[rowblk, full((N, NCLASS)),
                  pl.BlockSpec((RT, NCLASS), lambda i: (i, 0)),
                  colblk, colblk, full((1, NCLASS)), colblk,
                  full((1, N)), full((1, N))],
        out_specs=(pl.BlockSpec((RT, NCLASS), lambda i: (i, 0)),
                   pl.BlockSpec((1, 1), lambda i: (0, 0))),
        out_shape=(jax.ShapeDtypeStruct((N, NCLASS), f32),
                   jax.ShapeDtypeStruct((1, 1), f32)),
    )(adj, zm, z1, mask, dinv2, jnp.reshape(gb1, (1, NCLASS)), kappa,
      g_r, _row(gamma2))

    return out, jnp.reshape(curv, ())


# speedup vs baseline: 1.5655x; 1.5655x over previous
import jax
import jax.numpy as jnp
from jax.experimental import pallas as pl
from jax.experimental.pallas import tpu as pltpu

N = 2048
NFEAT = 256
NHID = 128
NCLASS = 16
RT = 256
NRT = N // RT
bf16 = jnp.bfloat16


def _prep_kernel(x_ref, cW1_ref, cb1_ref, cW2r_ref, cb2_ref,
                 fW1_ref, fb1_ref, fW2r_ref, fb2_ref, gW0_ref,
                 kappa_ref, f_ref, h0_ref):
    x = x_ref[...]
    hc = jax.nn.relu(jnp.dot(x, cW1_ref[...],
                             preferred_element_type=jnp.float32) + cb1_ref[...])
    kappa_ref[...] = jax.nn.sigmoid(
        jnp.sum(hc * cW2r_ref[...], axis=1, keepdims=True) + cb2_ref[...])
    hf = jax.nn.relu(jnp.dot(x, fW1_ref[...],
                             preferred_element_type=jnp.float32) + fb1_ref[...])
    f_ref[...] = jnp.sum(hf * fW2r_ref[...], axis=1, keepdims=True) + fb2_ref[...]
    h0_ref[...] = jnp.dot(x, gW0_ref[...], preferred_element_type=jnp.float32)


def _main_kernel(adj_ref, wts_ref, kc_ref, kr_ref, krm_ref,
                 f_ref, v3_ref, h0_ref, gb0_ref, gW1_ref, gb1_ref,
                 out_ref, curv_ref,
                 wa_s, mask_s, g_s, g2_s, d1_s, d2_s,
                 v2_s, g0_s, z1_s, zm_s):
    p = pl.program_id(0)
    i = pl.program_id(1)
    rows = pl.ds(i * RT, RT)
    f32 = jnp.float32

    @pl.when(p == 0)
    def _phase0():
        adj = adj_ref[...]
        wab = (adj * jax.nn.sigmoid(wts_ref[...])).astype(bf16)
        wa_s[rows, :] = wab
        s = jnp.dot(wab, v3_ref[...], preferred_element_type=f32)
        s0, s1, s2 = s[:, 0:1], s[:, 1:2], s[:, 2:3]
        sa = jnp.sum(adj, axis=1, keepdims=True)
        fc = f_ref[rows, :]
        g = 0.5 * (s2 - 2.0 * fc * s1 + fc * fc * s0)
        df = s1 - fc * s0
        g_s[rows, :] = g
        v2_s[rows, 0:1] = g
        v2_s[rows, 1:2] = df
        v2_s[rows, 2:3] = fc * df
        v2_s[rows, 3:4] = s0
        v2_s[rows, 4:5] = s1
        d1 = jax.lax.rsqrt(sa + 1.0)
        d1_s[rows, :] = d1
        g0_s[rows, :] = d1 * h0_ref[rows, :]
        kc = kc_ref[rows, :]
        kr = kr_ref[...]
        jj = jax.lax.broadcasted_iota(jnp.int32, (RT, N), 1)
        ii = jax.lax.broadcasted_iota(jnp.int32, (RT, N), 0) + i * RT
        cond = jnp.logical_or(kr > kc,
                              jnp.logical_and(kr == kc, jj < ii))
        rank = jnp.sum(cond.astype(f32), axis=1, keepdims=True)
        mask_s[rows, :] = (rank >= krm_ref[...]).astype(f32)

    @pl.when(p == 1)
    def _phase1():
        wab = wa_s[rows, :]
        adjb = (wab > 0).astype(bf16)
        t = jnp.dot(wab, v2_s[...].astype(bf16), preferred_element_type=f32)
        t_g, t_d, t_fd = t[:, 0:1], t[:, 1:2], t[:, 2:3]
        v2 = v2_s[rows, :]
        gc, dfc, s0, s1 = v2[:, 0:1], v2[:, 1:2], v2[:, 3:4], v2[:, 4:5]
        fc = f_ref[rows, :]
        delta_gamma = t_g - gc * s0
        gfd = 0.5 * (t_fd - fc * t_d - dfc * s1 + fc * dfc * s0)
        g2_s[rows, :] = 0.5 * delta_gamma - gfd
        sm = jnp.dot(adjb, mask_s[...].astype(bf16), preferred_element_type=f32)
        mc = mask_s[rows, :]
        d2 = jax.lax.rsqrt(mc * sm + 1.0)
        d2_s[rows, :] = d2
        y0 = jnp.dot(adjb, g0_s[...].astype(bf16), preferred_element_type=f32)
        h1 = jax.nn.relu(d1_s[rows, :] * (y0 + g0_s[rows, :]) + gb0_ref[...])
        z1 = d2 * jnp.dot(h1, gW1_ref[...], preferred_element_type=f32)
        z1_s[rows, :] = z1
        zm_s[rows, :] = (mc * z1).astype(bf16)

    @pl.when(p == 2)
    def _phase2():
        adjb = (wa_s[rows, :] > 0).astype(bf16)
        y = jnp.dot(adjb, zm_s[...], preferred_element_type=f32)
        agg = mask_s[rows, :] * y + z1_s[rows, :]
        h2 = jax.nn.relu(d2_s[rows, :] * agg + gb1_ref[...])
        m = jnp.max(h2, axis=1, keepdims=True)
        e = h2 - m
        out_ref[...] = e - jnp.log(jnp.sum(jnp.exp(e), axis=1, keepdims=True))
        part = jnp.reshape(
            jnp.sum(jax.nn.relu(g_s[rows, :] * kr_ref[...] - g2_s[rows, :])),
            (1, 1))

        @pl.when(i == 0)
        def _():
            curv_ref[...] = part

        @pl.when(i > 0)
        def _():
            curv_ref[...] += part


def kernel(x, adj, weights, cW1, cb1, cW2, cb2, fW1, fb1, fW2, fb2,
           gW0, gb0, gW1, gb1, p):
    f32 = jnp.float32
    kappa, f, h0 = pl.pallas_call(
        _prep_kernel,
        out_shape=(jax.ShapeDtypeStruct((N, 1), f32),
                   jax.ShapeDtypeStruct((N, 1), f32),
                   jax.ShapeDtypeStruct((N, NHID), f32)),
    )(x, cW1, jnp.reshape(cb1, (1, NHID)), jnp.reshape(cW2, (1, NHID)),
      jnp.reshape(cb2, (1, 1)), fW1, jnp.reshape(fb1, (1, NHID)),
      jnp.reshape(fW2, (1, NHID)), jnp.reshape(fb2, (1, 1)), gW0)

    k_rm = jnp.floor_divide(N * p, 100).astype(f32).reshape(1, 1)
    ones = jnp.ones((N, 1), f32)
    v3 = jnp.concatenate([ones, f, f * f], axis=1).astype(bf16)
    frozen = lambda pp, ii: (jnp.where(pp == 0, ii, NRT - 1), 0)
    const = lambda shape: pl.BlockSpec(shape, lambda pp, ii: (0, 0))

    out, curv = pl.pallas_call(
        _main_kernel,
        grid=(3, NRT),
        in_specs=[pl.BlockSpec((RT, N), frozen),
                  pl.BlockSpec((RT, N), frozen),
                  const((N, 1)),
                  const((1, N)),
                  const((1, 1)),
                  const((N, 1)),
                  const((N, 3)),
                  const((N, NHID)),
                  const((1, NHID)),
                  const((NHID, NCLASS)),
                  const((1, NCLASS))],
        out_specs=(pl.BlockSpec((RT, NCLASS), lambda pp, ii: (ii, 0)),
                   pl.BlockSpec((1, 1), lambda pp, ii: (0, 0))),
        out_shape=(jax.ShapeDtypeStruct((N, NCLASS), f32),
                   jax.ShapeDtypeStruct((1, 1), f32)),
        scratch_shapes=[pltpu.VMEM((N, N), bf16),
                        pltpu.VMEM((N, 1), f32),
                        pltpu.VMEM((N, 1), f32),
                        pltpu.VMEM((N, 1), f32),
                        pltpu.VMEM((N, 1), f32),
                        pltpu.VMEM((N, 1), f32),
                        pltpu.VMEM((N, 8), f32),
                        pltpu.VMEM((N, NHID), f32),
                        pltpu.VMEM((N, NCLASS), f32),
                        pltpu.VMEM((N, NCLASS), bf16)],
        compiler_params=pltpu.CompilerParams(
            dimension_semantics=("arbitrary", "arbitrary")),
    )(adj, weights, kappa, jnp.reshape(kappa, (1, N)), k_rm, f, v3, h0,
      jnp.reshape(gb0, (1, NHID)), gW1, jnp.reshape(gb1, (1, NCLASS)))

    return out, jnp.reshape(curv, ())
